# Initial kernel scaffold; baseline (speedup 1.0000x reference)
import jax
import jax.numpy as jnp
from jax.experimental import pallas as pl


def kernel(x, x_str, edge_index, lin0_w, lin0_b, lin11_w, lin11_b, conv_w1, conv_w2, conv1_w1, conv1_w2, lins1_w, lins1_b, lin3_w, lin3_b):
    raise NotImplementedError("write your pallas kernel here")


# speedup vs baseline: 1.8842x; 1.8842x over previous
import functools

import numpy as np
import jax
import jax.numpy as jnp
from jax import lax
from jax.experimental import pallas as pl
from jax.experimental.pallas import tpu as pltpu

ALPHA = 0.4
THETA = 0.9

BN = 2000


def _init_body(x_ref, xs_ref, degp_ref, w0_ref, b0_ref, w11_ref, b11_ref,
               h0_ref, g0_ref, hs0_ref, gs0_ref, dinv_ref):
    deg = jnp.sum(degp_ref[:, :, 0], axis=0)
    dinv = jnp.where(deg > 0, lax.rsqrt(deg), 0.0)[:, None]
    h0 = jnp.maximum(
        jnp.dot(x_ref[...], w0_ref[...], preferred_element_type=jnp.float32)
        + b0_ref[...], 0.0)
    g0 = jnp.maximum(
        jnp.dot(xs_ref[...], w11_ref[...], preferred_element_type=jnp.float32)
        + b11_ref[...], 0.0)
    h0_ref[...] = h0
    g0_ref[...] = g0
    hs0_ref[...] = dinv * h0
    gs0_ref[...] = dinv * g0
    dinv_ref[...] = dinv


def _mk_init(n, c, k, f_in, f_str_p, h):
    grid = (n + BN - 1) // BN
    return pl.pallas_call(
        _init_body,
        grid=(grid,),
        in_specs=[
            pl.BlockSpec((BN, f_in), lambda i: (i, 0)),
            pl.BlockSpec((BN, f_str_p), lambda i: (i, 0)),
            pl.BlockSpec((c, BN, k), lambda i: (0, i, 0)),
            pl.BlockSpec((f_in, h), lambda i: (0, 0)),
            pl.BlockSpec((1, h), lambda i: (0, 0)),
            pl.BlockSpec((f_str_p, h), lambda i: (0, 0)),
            pl.BlockSpec((1, h), lambda i: (0, 0)),
        ],
        out_specs=[
            pl.BlockSpec((BN, h), lambda i: (i, 0)),
            pl.BlockSpec((BN, h), lambda i: (i, 0)),
            pl.BlockSpec((BN, h), lambda i: (i, 0)),
            pl.BlockSpec((BN, h), lambda i: (i, 0)),
            pl.BlockSpec((BN, 1), lambda i: (i, 0)),
        ],
        out_shape=[
            jax.ShapeDtypeStruct((n, h), jnp.float32),
            jax.ShapeDtypeStruct((n, h), jnp.float32),
            jax.ShapeDtypeStruct((n, h), jnp.float32),
            jax.ShapeDtypeStruct((n, h), jnp.float32),
            jax.ShapeDtypeStruct((n, 1), jnp.float32),
        ],
    )


def _combine_body(php_ref, pgp_ref, dinv_ref, h0_ref, g0_ref,
                  w1h_ref, w2h_ref, w1g_ref, w2g_ref,
                  h_ref, hs_ref, g_ref, gs_ref, *, beta):
    dinv = dinv_ref[...]

    def branch(pp_ref, z0_ref, w1_ref, w2_ref):
        p = jnp.sum(pp_ref[...], axis=0)
        hp = (1.0 - ALPHA) * dinv * p
        z0a = ALPHA * z0_ref[...]
        out = ((1.0 - beta) * hp
               + beta * jnp.dot(hp, w1_ref[0], preferred_element_type=jnp.float32)
               + (1.0 - beta) * z0a
               + beta * jnp.dot(z0a, w2_ref[0], preferred_element_type=jnp.float32))
        return jnp.maximum(out, 0.0)

    hh = branch(php_ref, h0_ref, w1h_ref, w2h_ref)
    gg = branch(pgp_ref, g0_ref, w1g_ref, w2g_ref)
    h_ref[...] = hh
    hs_ref[...] = dinv * hh
    g_ref[...] = gg
    gs_ref[...] = dinv * gg


def _final_body(php_ref, pgp_ref, dinv_ref, h0_ref, g0_ref,
                w1h_ref, w2h_ref, w1g_ref, w2g_ref,
                wzh_ref, bzh_ref, wzg_ref, bzg_ref,
                z_ref, z1_ref, *, beta):
    dinv = dinv_ref[...]

    def branch(pp_ref, z0_ref, w1_ref, w2_ref):
        p = jnp.sum(pp_ref[...], axis=0)
        hp = (1.0 - ALPHA) * dinv * p
        z0a = ALPHA * z0_ref[...]
        out = ((1.0 - beta) * hp
               + beta * jnp.dot(hp, w1_ref[0], preferred_element_type=jnp.float32)
               + (1.0 - beta) * z0a
               + beta * jnp.dot(z0a, w2_ref[0], preferred_element_type=jnp.float32))
        return jnp.maximum(out, 0.0)

    hh = branch(php_ref, h0_ref, w1h_ref, w2h_ref)
    gg = branch(pgp_ref, g0_ref, w1g_ref, w2g_ref)
    z_ref[...] = jnp.dot(hh, wzh_ref[...], preferred_element_type=jnp.float32) + bzh_ref[...]
    z1_ref[...] = jnp.dot(gg, wzg_ref[...], preferred_element_type=jnp.float32) + bzg_ref[...]


def _layer_specs(n, c, h, layer):
    return [
        pl.BlockSpec((c, BN, h), lambda i: (0, i, 0)),
        pl.BlockSpec((c, BN, h), lambda i: (0, i, 0)),
        pl.BlockSpec((BN, 1), lambda i: (i, 0)),
        pl.BlockSpec((BN, h), lambda i: (i, 0)),
        pl.BlockSpec((BN, h), lambda i: (i, 0)),
        pl.BlockSpec((1, h, h), lambda i, l=layer: (l, 0, 0)),
        pl.BlockSpec((1, h, h), lambda i, l=layer: (l, 0, 0)),
        pl.BlockSpec((1, h, h), lambda i, l=layer: (l, 0, 0)),
        pl.BlockSpec((1, h, h), lambda i, l=layer: (l, 0, 0)),
    ]


def _mk_combine(n, c, h, layer, beta):
    grid = (n + BN - 1) // BN
    return pl.pallas_call(
        functools.partial(_combine_body, beta=beta),
        grid=(grid,),
        in_specs=_layer_specs(n, c, h, layer),
        out_specs=[pl.BlockSpec((BN, h), lambda i: (i, 0))] * 4,
        out_shape=[jax.ShapeDtypeStruct((n, h), jnp.float32)] * 4,
    )


def _mk_final(n, c, h, layer, beta):
    grid = (n + BN - 1) // BN
    return pl.pallas_call(
        functools.partial(_final_body, beta=beta),
        grid=(grid,),
        in_specs=_layer_specs(n, c, h, layer) + [
            pl.BlockSpec((h, 1), lambda i: (0, 0)),
            pl.BlockSpec((1, 1), lambda i: (0, 0)),
            pl.BlockSpec((h, 1), lambda i: (0, 0)),
            pl.BlockSpec((1, 1), lambda i: (0, 0)),
        ],
        out_specs=[pl.BlockSpec((BN, 1), lambda i: (i, 0))] * 2,
        out_shape=[jax.ShapeDtypeStruct((n, 1), jnp.float32)] * 2,
    )


def kernel(x, x_str, edge_index, lin0_w, lin0_b, lin11_w, lin11_b,
           conv_w1, conv_w2, conv1_w1, conv1_w2, lins1_w, lins1_b,
           lin3_w, lin3_b):
    n, f_in = x.shape
    f_str = x_str.shape[1]
    h = lin0_w.shape[1]
    num_layers = conv_w1.shape[0]
    e = edge_index.shape[1]

    row = edge_index[0].astype(jnp.int32)
    col = edge_index[1].astype(jnp.int32)

    f_str_p = ((f_str + 7) // 8) * 8
    xs = jnp.pad(x_str, ((0, 0), (0, f_str_p - f_str)))
    w11 = jnp.pad(lin11_w, ((0, f_str_p - f_str), (0, 0)))

    deg = jax.ops.segment_sum(jnp.ones((e,), jnp.float32), col, num_segments=n)
    degp = deg.reshape(1, n, 1)

    def prop(t):
        return jax.ops.segment_sum(t[row], col, num_segments=n).reshape(1, n, h)

    c = degp.shape[0]
    init = _mk_init(n, c, degp.shape[2], f_in, f_str_p, h)
    h0, g0, hs, gs, dinv = init(x, xs, degp, lin0_w, lin0_b.reshape(1, h),
                                w11, lin11_b.reshape(1, h))

    for i in range(num_layers):
        beta = float(np.log(THETA / (i + 1) + 1.0))
        ph = prop(hs)
        pg = prop(gs)
        if i < num_layers - 1:
            comb = _mk_combine(n, 1, h, i, beta)
            _h, hs, _g, gs = comb(ph, pg, dinv, h0, g0,
                                  conv_w1, conv_w2, conv1_w1, conv1_w2)
        else:
            fin = _mk_final(n, 1, h, i, beta)
            z, z1 = fin(ph, pg, dinv, h0, g0,
                        conv_w1, conv_w2, conv1_w1, conv1_w2,
                        lins1_w, lins1_b.reshape(1, 1),
                        lin3_w, lin3_b.reshape(1, 1))
    return (z, z1)


# speedup vs baseline: 12.4529x; 6.6091x over previous
import functools

import numpy as np
import jax
import jax.numpy as jnp
from jax import lax
from jax.experimental import pallas as pl
from jax.experimental.pallas import tpu as pltpu
from jax.experimental.pallas import tpu_sc as plsc

ALPHA = 0.4
THETA = 0.9

BN = 2000

_NC, _NS = 2, 16
_NW = _NC * _NS
_CH = 128
_KD = 16


def _n_acc(n):
    return (n // 128 + 1) * 128


def _sc_prop(n, h, k):
    n_acc = _n_acc(n)
    rpt = n_acc // _NS
    mesh = plsc.VectorSubcoreMesh(core_axis_name="c", subcore_axis_name="s")

    @functools.partial(
        pl.kernel,
        out_type=[jax.ShapeDtypeStruct((_NC, n_acc, h), jnp.float32),
                  jax.ShapeDtypeStruct((_NC, n_acc, h), jnp.float32)],
        mesh=mesh,
        compiler_params=pltpu.CompilerParams(use_tc_tiling_on_sc=False),
        scratch_types=[
            pltpu.VMEM((k, _CH), jnp.int32),
            pltpu.VMEM((k, _CH), jnp.int32),
            pltpu.VMEM((_CH, h), jnp.float32),
            pltpu.VMEM((_CH, h), jnp.float32),
            pltpu.VMEM_SHARED((n_acc, h), jnp.float32),
            pltpu.VMEM_SHARED((n_acc, h), jnp.float32),
            pltpu.SemaphoreType.DMA,
            pltpu.SemaphoreType.DMA,
            pltpu.SemaphoreType.DMA,
            pltpu.SemaphoreType.DMA,
        ],
    )
    def prop2(hs_hbm, gs_hbm, rowp, colp, zeros_hbm, outh, outg,
              row_v, col_v, rbh, rbg, acch, accg, gsh, gsg, ssh, ssg):
        cid = lax.axis_index("c")
        sid = lax.axis_index("s")
        wid = cid * _NS + sid
        pltpu.sync_copy(rowp.at[wid], row_v)
        pltpu.sync_copy(colp.at[wid], col_v)
        zb = sid * rpt
        pltpu.sync_copy(zeros_hbm.at[pl.ds(zb, rpt)], acch.at[pl.ds(zb, rpt)])
        pltpu.sync_copy(zeros_hbm.at[pl.ds(zb, rpt)], accg.at[pl.ds(zb, rpt)])
        plsc.subcore_barrier()

        def body(j, carry):
            dh = pltpu.async_copy(hs_hbm.at[row_v.at[j]], rbh, gsh)
            dg = pltpu.async_copy(gs_hbm.at[row_v.at[j]], rbg, gsg)
            dh.wait()
            sh = pltpu.async_copy(rbh, acch.at[col_v.at[j]], ssh, add=True)
            dg.wait()
            sg = pltpu.async_copy(rbg, accg.at[col_v.at[j]], ssg, add=True)
            sh.wait()
            sg.wait()
            return carry

        lax.fori_loop(0, k, body, 0)
        plsc.subcore_barrier()
        pltpu.sync_copy(acch.at[pl.ds(zb, rpt)], outh.at[cid, pl.ds(zb, rpt)])
        pltpu.sync_copy(accg.at[pl.ds(zb, rpt)], outg.at[cid, pl.ds(zb, rpt)])

    return prop2


def _sc_deg(n, k):
    n_acc = _n_acc(n)
    rpt = n_acc // _NS
    mesh = plsc.VectorSubcoreMesh(core_axis_name="c", subcore_axis_name="s")

    @functools.partial(
        pl.kernel,
        out_type=jax.ShapeDtypeStruct((_NC, n_acc, _KD), jnp.float32),
        mesh=mesh,
        compiler_params=pltpu.CompilerParams(use_tc_tiling_on_sc=False),
        scratch_types=[
            pltpu.VMEM((k, _CH), jnp.int32),
            pltpu.VMEM((_CH, _KD), jnp.float32),
            pltpu.VMEM_SHARED((n_acc, _KD), jnp.float32),
            pltpu.SemaphoreType.DMA,
        ],
    )
    def degk(colp, ones_hbm, zeros_hbm, outd, col_v, ones_v, accd, sem):
        cid = lax.axis_index("c")
        sid = lax.axis_index("s")
        wid = cid * _NS + sid
        pltpu.sync_copy(colp.at[wid], col_v)
        pltpu.sync_copy(ones_hbm, ones_v)
        zb = sid * rpt
        pltpu.sync_copy(zeros_hbm.at[pl.ds(zb, rpt)], accd.at[pl.ds(zb, rpt)])
        plsc.subcore_barrier()

        def body(j, carry):
            pltpu.async_copy(ones_v, accd.at[col_v.at[j]], sem, add=True).wait()
            return carry

        lax.fori_loop(0, k, body, 0)
        plsc.subcore_barrier()
        pltpu.sync_copy(accd.at[pl.ds(zb, rpt)], outd.at[cid, pl.ds(zb, rpt)])

    return degk


def _init_body(x_ref, xs_ref, degp_ref, w0_ref, b0_ref, w11_ref, b11_ref,
               h0_ref, g0_ref, hs0_ref, gs0_ref, dinv_ref):
    deg = jnp.sum(degp_ref[:, :, 0], axis=0)
    dinv = jnp.where(deg > 0, lax.rsqrt(deg), 0.0)[:, None]
    h0 = jnp.maximum(
        jnp.dot(x_ref[...], w0_ref[...], preferred_element_type=jnp.float32)
        + b0_ref[...], 0.0)
    g0 = jnp.maximum(
        jnp.dot(xs_ref[...], w11_ref[...], preferred_element_type=jnp.float32)
        + b11_ref[...], 0.0)
    h0_ref[...] = h0
    g0_ref[...] = g0
    hs0_ref[...] = dinv * h0
    gs0_ref[...] = dinv * g0
    dinv_ref[...] = dinv


def _mk_init(n, c, k, f_in, f_str_p, h):
    grid = (n + BN - 1) // BN
    return pl.pallas_call(
        _init_body,
        grid=(grid,),
        in_specs=[
            pl.BlockSpec((BN, f_in), lambda i: (i, 0)),
            pl.BlockSpec((BN, f_str_p), lambda i: (i, 0)),
            pl.BlockSpec((c, BN, k), lambda i: (0, i, 0)),
            pl.BlockSpec((f_in, h), lambda i: (0, 0)),
            pl.BlockSpec((1, h), lambda i: (0, 0)),
            pl.BlockSpec((f_str_p, h), lambda i: (0, 0)),
            pl.BlockSpec((1, h), lambda i: (0, 0)),
        ],
        out_specs=[
            pl.BlockSpec((BN, h), lambda i: (i, 0)),
            pl.BlockSpec((BN, h), lambda i: (i, 0)),
            pl.BlockSpec((BN, h), lambda i: (i, 0)),
            pl.BlockSpec((BN, h), lambda i: (i, 0)),
            pl.BlockSpec((BN, 1), lambda i: (i, 0)),
        ],
        out_shape=[
            jax.ShapeDtypeStruct((n, h), jnp.float32),
            jax.ShapeDtypeStruct((n, h), jnp.float32),
            jax.ShapeDtypeStruct((n, h), jnp.float32),
            jax.ShapeDtypeStruct((n, h), jnp.float32),
            jax.ShapeDtypeStruct((n, 1), jnp.float32),
        ],
    )


def _combine_body(php_ref, pgp_ref, dinv_ref, h0_ref, g0_ref,
                  w1h_ref, w2h_ref, w1g_ref, w2g_ref,
                  h_ref, hs_ref, g_ref, gs_ref, *, beta):
    dinv = dinv_ref[...]

    def branch(pp_ref, z0_ref, w1_ref, w2_ref):
        p = jnp.sum(pp_ref[...], axis=0)
        hp = (1.0 - ALPHA) * dinv * p
        z0a = ALPHA * z0_ref[...]
        out = ((1.0 - beta) * hp
               + beta * jnp.dot(hp, w1_ref[0], preferred_element_type=jnp.float32)
               + (1.0 - beta) * z0a
               + beta * jnp.dot(z0a, w2_ref[0], preferred_element_type=jnp.float32))
        return jnp.maximum(out, 0.0)

    hh = branch(php_ref, h0_ref, w1h_ref, w2h_ref)
    gg = branch(pgp_ref, g0_ref, w1g_ref, w2g_ref)
    h_ref[...] = hh
    hs_ref[...] = dinv * hh
    g_ref[...] = gg
    gs_ref[...] = dinv * gg


def _final_body(php_ref, pgp_ref, dinv_ref, h0_ref, g0_ref,
                w1h_ref, w2h_ref, w1g_ref, w2g_ref,
                wzh_ref, bzh_ref, wzg_ref, bzg_ref,
                z_ref, z1_ref, *, beta):
    dinv = dinv_ref[...]

    def branch(pp_ref, z0_ref, w1_ref, w2_ref):
        p = jnp.sum(pp_ref[...], axis=0)
        hp = (1.0 - ALPHA) * dinv * p
        z0a = ALPHA * z0_ref[...]
        out = ((1.0 - beta) * hp
               + beta * jnp.dot(hp, w1_ref[0], preferred_element_type=jnp.float32)
               + (1.0 - beta) * z0a
               + beta * jnp.dot(z0a, w2_ref[0], preferred_element_type=jnp.float32))
        return jnp.maximum(out, 0.0)

    hh = branch(php_ref, h0_ref, w1h_ref, w2h_ref)
    gg = branch(pgp_ref, g0_ref, w1g_ref, w2g_ref)
    z_ref[...] = jnp.dot(hh, wzh_ref[...], preferred_element_type=jnp.float32) + bzh_ref[...]
    z1_ref[...] = jnp.dot(gg, wzg_ref[...], preferred_element_type=jnp.float32) + bzg_ref[...]


def _layer_specs(n, c, h, layer):
    return [
        pl.BlockSpec((c, BN, h), lambda i: (0, i, 0)),
        pl.BlockSpec((c, BN, h), lambda i: (0, i, 0)),
        pl.BlockSpec((BN, 1), lambda i: (i, 0)),
        pl.BlockSpec((BN, h), lambda i: (i, 0)),
        pl.BlockSpec((BN, h), lambda i: (i, 0)),
        pl.BlockSpec((1, h, h), lambda i, l=layer: (l, 0, 0)),
        pl.BlockSpec((1, h, h), lambda i, l=layer: (l, 0, 0)),
        pl.BlockSpec((1, h, h), lambda i, l=layer: (l, 0, 0)),
        pl.BlockSpec((1, h, h), lambda i, l=layer: (l, 0, 0)),
    ]


def _mk_combine(n, c, h, layer, beta):
    grid = (n + BN - 1) // BN
    return pl.pallas_call(
        functools.partial(_combine_body, beta=beta),
        grid=(grid,),
        in_specs=_layer_specs(n, c, h, layer),
        out_specs=[pl.BlockSpec((BN, h), lambda i: (i, 0))] * 4,
        out_shape=[jax.ShapeDtypeStruct((n, h), jnp.float32)] * 4,
    )


def _mk_final(n, c, h, layer, beta):
    grid = (n + BN - 1) // BN
    return pl.pallas_call(
        functools.partial(_final_body, beta=beta),
        grid=(grid,),
        in_specs=_layer_specs(n, c, h, layer) + [
            pl.BlockSpec((h, 1), lambda i: (0, 0)),
            pl.BlockSpec((1, 1), lambda i: (0, 0)),
            pl.BlockSpec((h, 1), lambda i: (0, 0)),
            pl.BlockSpec((1, 1), lambda i: (0, 0)),
        ],
        out_specs=[pl.BlockSpec((BN, 1), lambda i: (i, 0))] * 2,
        out_shape=[jax.ShapeDtypeStruct((n, 1), jnp.float32)] * 2,
    )


def kernel(x, x_str, edge_index, lin0_w, lin0_b, lin11_w, lin11_b,
           conv_w1, conv_w2, conv1_w1, conv1_w2, lins1_w, lins1_b,
           lin3_w, lin3_b):
    n, f_in = x.shape
    f_str = x_str.shape[1]
    h = lin0_w.shape[1]
    num_layers = conv_w1.shape[0]
    e = edge_index.shape[1]

    row = edge_index[0].astype(jnp.int32)
    col = edge_index[1].astype(jnp.int32)

    f_str_p = ((f_str + 7) // 8) * 8
    xs = jnp.pad(x_str, ((0, 0), (0, f_str_p - f_str)))
    w11 = jnp.pad(lin11_w, ((0, f_str_p - f_str), (0, 0)))

    k_ch = -(-e // (_NW * _CH))
    e_pad = _NW * k_ch * _CH
    rowp = jnp.concatenate(
        [row, jnp.zeros((e_pad - e,), jnp.int32)]).reshape(_NW, k_ch, _CH)
    colp = jnp.concatenate(
        [col, jnp.full((e_pad - e,), n, jnp.int32)]).reshape(_NW, k_ch, _CH)
    zeros_nh = jnp.zeros((_n_acc(n), h), jnp.float32)
    zeros_nd = jnp.zeros((_n_acc(n), _KD), jnp.float32)
    ones_d = jnp.ones((_CH, _KD), jnp.float32)

    degp = _sc_deg(n, k_ch)(colp, ones_d, zeros_nd)
    prop2 = _sc_prop(n, h, k_ch)

    c = degp.shape[0]
    init = _mk_init(n, c, degp.shape[2], f_in, f_str_p, h)
    h0, g0, hs, gs, dinv = init(x, xs, degp, lin0_w, lin0_b.reshape(1, h),
                                w11, lin11_b.reshape(1, h))

    for i in range(num_layers):
        beta = float(np.log(THETA / (i + 1) + 1.0))
        ph, pg = prop2(hs, gs, rowp, colp, zeros_nh)
        if i < num_layers - 1:
            comb = _mk_combine(n, _NC, h, i, beta)
            _h, hs, _g, gs = comb(ph, pg, dinv, h0, g0,
                                  conv_w1, conv_w2, conv1_w1, conv1_w2)
        else:
            fin = _mk_final(n, _NC, h, i, beta)
            z, z1 = fin(ph, pg, dinv, h0, g0,
                        conv_w1, conv_w2, conv1_w1, conv1_w2,
                        lins1_w, lins1_b.reshape(1, 1),
                        lin3_w, lin3_b.reshape(1, 1))
    return (z, z1)
